# baseline (speedup 1.0000x reference)
import jax
import jax.numpy as jnp
import numpy as np
from jax.experimental import pallas as pl
from jax.experimental.pallas import tpu as pltpu

S = 14
STEP = 1.0 / S
LAMBDA_COORD = 7.0
LAMBDA_NOOBJ = 0.2
LAMBDA_CLS = 1.5
BATCH = 1024
CELLS = S * S
CH = 30

BL = 128
GRID = BATCH // BL


def _atan_pos(x):
    big = x > 2.414213562373095
    mid = x > 0.41421356237309503
    y = jnp.where(big, np.float32(np.pi / 2),
                  jnp.where(mid, np.float32(np.pi / 4), np.float32(0.0)))
    xr = jnp.where(big, -1.0 / x, jnp.where(mid, (x - 1.0) / (x + 1.0), x))
    z = xr * xr
    p = ((((8.05374449538e-2 * z - 1.38776856032e-1) * z
           + 1.99777106478e-1) * z - 3.33329491539e-1) * z * xr + xr)
    return y + p


def _iou(b1, b2):
    x1 = jnp.maximum(b1[0], b2[0])
    y1 = jnp.maximum(b1[1], b2[1])
    x2 = jnp.minimum(b1[2], b2[2])
    y2 = jnp.minimum(b1[3], b2[3])
    inter = jnp.maximum(x2 - x1, 0.0) * jnp.maximum(y2 - y1, 0.0)
    area1 = (b1[2] - b1[0]) * (b1[3] - b1[1])
    area2 = (b2[2] - b2[0]) * (b2[3] - b2[1])
    union = area1 + area2 - inter + 1e-06
    return inter / union


def _ciou(b1, b2, iou):
    cx1 = (b1[0] + b1[2]) / 2
    cy1 = (b1[1] + b1[3]) / 2
    cx2 = (b2[0] + b2[2]) / 2
    cy2 = (b2[1] + b2[3]) / 2
    center_dist = (cx1 - cx2) ** 2 + (cy1 - cy2) ** 2
    x_c1 = jnp.minimum(b1[0], b2[0])
    y_c1 = jnp.minimum(b1[1], b2[1])
    x_c2 = jnp.maximum(b1[2], b2[2])
    y_c2 = jnp.maximum(b1[3], b2[3])
    outer_diag = (x_c2 - x_c1) ** 2 + (y_c2 - y_c1) ** 2 + 1e-06
    w1 = jnp.maximum(b1[2] - b1[0], 1e-06)
    h1 = jnp.maximum(b1[3] - b1[1], 1e-06)
    w2 = jnp.maximum(b2[2] - b2[0], 1e-06)
    h2 = jnp.maximum(b2[3] - b2[1], 1e-06)
    v = 4.0 / np.pi ** 2 * (_atan_pos(w2 / h2) - _atan_pos(w1 / h1)) ** 2
    alpha = v / (1.0 - iou + v + 1e-06)
    ciou = iou - center_dist / outer_diag - alpha * v
    scale = jnp.maximum(2.0 - w2 * h2, 1.0)
    return (1.0 - ciou) * scale


def _xyxy(x, y, w, h, ii, jj):
    cx = (x + ii) * STEP
    cy = (y + jj) * STEP
    return (cx - w / 2, cy - h / 2, cx + w / 2, cy + h / 2)


def _loss_kernel(p_ref, t_ref, ph_ref, th_ref, o_ref, *scratch):
    pv = scratch[:10]
    tv = scratch[10:20]
    sem = scratch[20]
    step = pl.program_id(0)

    copies = []
    for c in range(10):
        copies.append(pltpu.make_async_copy(
            ph_ref.at[:, c, pl.ds(step * BL, BL)], pv[c], sem))
        copies.append(pltpu.make_async_copy(
            th_ref.at[:, c, pl.ds(step * BL, BL)], tv[c], sem))
    for cp in copies:
        cp.start()

    bsum = jnp.zeros((CELLS, BL), jnp.float32)
    for lo, hi in ((8, 16), (16, 24), (24, 30)):
        ps = p_ref[:, lo:hi, :]
        ts = t_ref[:, lo:hi, :]
        lp = jnp.log(jnp.maximum(ps, 1e-12))
        lm = jnp.log(1.0 - ps)
        b = -(lm + ts * (lp - lm))
        if lo == 8:
            ch_id = jax.lax.broadcasted_iota(jnp.int32, (CELLS, hi - lo, BL), 1)
            b = jnp.where(ch_id >= 2, b, 0.0)
        bsum = bsum + jnp.sum(b, axis=1)

    for cp in copies:
        cp.wait()

    def pch(c):
        return pv[c][...]

    def tch(c):
        return tv[c][...]

    k = jax.lax.broadcasted_iota(jnp.int32, (CELLS, BL), 0)
    ii = jax.lax.rem(k, S).astype(jnp.float32)
    jj = (k // S).astype(jnp.float32)

    pb0 = _xyxy(pch(0), pch(1), pch(2), pch(3), ii, jj)
    pb1 = _xyxy(pch(5), pch(6), pch(7), pch(8), ii, jj)
    tb0 = _xyxy(tch(0), tch(1), tch(2), tch(3), ii, jj)
    tb1 = _xyxy(tch(5), tch(6), tch(7), tch(8), ii, jj)

    iou0 = _iou(pb0, tb0)
    iou1 = _iou(pb1, tb1)
    max0 = iou0 >= iou1

    tc0 = tch(4)
    tc1 = tch(9)
    obj00 = tc0 > 0
    sig = tc1 > 0

    cse0 = (pch(4) - tc0) ** 2
    cse1 = (pch(9) - tc1) ** 2
    cse_tot = jnp.sum(cse0 + cse1)

    obj_cell = (jnp.where(obj00 & (~sig | max0), cse0, 0.0)
                + jnp.where(sig & ~max0, cse1, 0.0))
    obj_sum = jnp.sum(obj_cell)

    ciou0 = _ciou(pb0, tb0, iou0)
    ciou1 = _ciou(pb1, tb1, iou1)
    bbox_cell = (jnp.where(sig & max0 & obj00, ciou0, 0.0)
                 + jnp.where(sig & ~max0, ciou1, 0.0))
    bbox_sum = jnp.sum(bbox_cell)

    cls_sum = jnp.sum(jnp.where(sig, bsum, 0.0))

    partial = (obj_sum + LAMBDA_NOOBJ * (cse_tot - obj_sum)
               + LAMBDA_COORD * bbox_sum + LAMBDA_CLS * cls_sum)

    prev = jnp.where(step == 0, jnp.zeros((1, 1), jnp.float32), o_ref[...])
    total = prev + partial
    o_ref[...] = jnp.where(step == GRID - 1, total / BATCH, total)


def kernel(pred, target):
    pt = jnp.transpose(pred, (1, 2, 3, 0)).reshape(CELLS, CH, BATCH)
    tt = jnp.transpose(target, (1, 2, 3, 0)).reshape(CELLS, CH, BATCH)
    out = pl.pallas_call(
        _loss_kernel,
        grid=(GRID,),
        in_specs=[
            pl.BlockSpec((CELLS, CH, BL), lambda i: (0, 0, i)),
            pl.BlockSpec((CELLS, CH, BL), lambda i: (0, 0, i)),
            pl.BlockSpec(memory_space=pl.ANY),
            pl.BlockSpec(memory_space=pl.ANY),
        ],
        out_specs=pl.BlockSpec((1, 1), lambda i: (0, 0)),
        out_shape=jax.ShapeDtypeStruct((1, 1), jnp.float32),
        scratch_shapes=([pltpu.VMEM((CELLS, BL), jnp.float32)] * 20
                        + [pltpu.SemaphoreType.DMA]),
    )(pt, tt, pt, tt)
    return out[0, 0]


# speedup vs baseline: 1.1383x; 1.1383x over previous
import jax
import jax.numpy as jnp
import numpy as np
from jax.experimental import pallas as pl
from jax.experimental.pallas import tpu as pltpu

S = 14
STEP = 1.0 / S
LAMBDA_COORD = 7.0
LAMBDA_NOOBJ = 0.2
LAMBDA_CLS = 1.5
BATCH = 1024
CELLS = S * S
CH = 30

BL = 256
GRID = BATCH // BL


def _atan_pos(x):
    big = x > 2.414213562373095
    mid = x > 0.41421356237309503
    y = jnp.where(big, np.float32(np.pi / 2),
                  jnp.where(mid, np.float32(np.pi / 4), np.float32(0.0)))
    xr = jnp.where(big, -1.0 / x, jnp.where(mid, (x - 1.0) / (x + 1.0), x))
    z = xr * xr
    p = ((((8.05374449538e-2 * z - 1.38776856032e-1) * z
           + 1.99777106478e-1) * z - 3.33329491539e-1) * z * xr + xr)
    return y + p


def _iou(b1, b2):
    x1 = jnp.maximum(b1[0], b2[0])
    y1 = jnp.maximum(b1[1], b2[1])
    x2 = jnp.minimum(b1[2], b2[2])
    y2 = jnp.minimum(b1[3], b2[3])
    inter = jnp.maximum(x2 - x1, 0.0) * jnp.maximum(y2 - y1, 0.0)
    area1 = (b1[2] - b1[0]) * (b1[3] - b1[1])
    area2 = (b2[2] - b2[0]) * (b2[3] - b2[1])
    union = area1 + area2 - inter + 1e-06
    return inter / union


def _ciou(b1, b2, iou):
    cx1 = (b1[0] + b1[2]) / 2
    cy1 = (b1[1] + b1[3]) / 2
    cx2 = (b2[0] + b2[2]) / 2
    cy2 = (b2[1] + b2[3]) / 2
    center_dist = (cx1 - cx2) ** 2 + (cy1 - cy2) ** 2
    x_c1 = jnp.minimum(b1[0], b2[0])
    y_c1 = jnp.minimum(b1[1], b2[1])
    x_c2 = jnp.maximum(b1[2], b2[2])
    y_c2 = jnp.maximum(b1[3], b2[3])
    outer_diag = (x_c2 - x_c1) ** 2 + (y_c2 - y_c1) ** 2 + 1e-06
    w1 = jnp.maximum(b1[2] - b1[0], 1e-06)
    h1 = jnp.maximum(b1[3] - b1[1], 1e-06)
    w2 = jnp.maximum(b2[2] - b2[0], 1e-06)
    h2 = jnp.maximum(b2[3] - b2[1], 1e-06)
    v = 4.0 / np.pi ** 2 * (_atan_pos(w2 / h2) - _atan_pos(w1 / h1)) ** 2
    alpha = v / (1.0 - iou + v + 1e-06)
    ciou = iou - center_dist / outer_diag - alpha * v
    scale = jnp.maximum(2.0 - w2 * h2, 1.0)
    return (1.0 - ciou) * scale


def _xyxy(x, y, w, h, ii, jj):
    cx = (x + ii) * STEP
    cy = (y + jj) * STEP
    return (cx - w / 2, cy - h / 2, cx + w / 2, cy + h / 2)


def _loss_kernel(p_ref, t_ref, ph_ref, th_ref, o_ref, *scratch):
    pv = scratch[:10]
    tv = scratch[10:20]
    sem = scratch[20]
    step = pl.program_id(0)

    copies = []
    for c in range(10):
        copies.append(pltpu.make_async_copy(
            ph_ref.at[:, c, pl.ds(step * BL, BL)], pv[c], sem))
        copies.append(pltpu.make_async_copy(
            th_ref.at[:, c, pl.ds(step * BL, BL)], tv[c], sem))
    for cp in copies:
        cp.start()

    p = p_ref[...]
    t = t_ref[...]
    ch_id = jax.lax.broadcasted_iota(jnp.int32, (CELLS, CH, BL), 1)
    lp = jnp.log(jnp.maximum(p, 1e-12))
    lm = jnp.log(1.0 - p)
    bce_m = jnp.where(ch_id >= 10, -(lm + t * (lp - lm)), 0.0)
    bsum = jnp.sum(bce_m, axis=1)

    for cp in copies:
        cp.wait()

    def pch(c):
        return pv[c][...]

    def tch(c):
        return tv[c][...]

    k = jax.lax.broadcasted_iota(jnp.int32, (CELLS, BL), 0)
    ii = jax.lax.rem(k, S).astype(jnp.float32)
    jj = (k // S).astype(jnp.float32)

    pb0 = _xyxy(pch(0), pch(1), pch(2), pch(3), ii, jj)
    pb1 = _xyxy(pch(5), pch(6), pch(7), pch(8), ii, jj)
    tb0 = _xyxy(tch(0), tch(1), tch(2), tch(3), ii, jj)
    tb1 = _xyxy(tch(5), tch(6), tch(7), tch(8), ii, jj)

    iou0 = _iou(pb0, tb0)
    iou1 = _iou(pb1, tb1)
    max0 = iou0 >= iou1

    tc0 = tch(4)
    tc1 = tch(9)
    obj00 = tc0 > 0
    sig = tc1 > 0

    cse0 = (pch(4) - tc0) ** 2
    cse1 = (pch(9) - tc1) ** 2
    cse_tot = jnp.sum(cse0 + cse1)

    obj_cell = (jnp.where(obj00 & (~sig | max0), cse0, 0.0)
                + jnp.where(sig & ~max0, cse1, 0.0))
    obj_sum = jnp.sum(obj_cell)

    ciou0 = _ciou(pb0, tb0, iou0)
    ciou1 = _ciou(pb1, tb1, iou1)
    bbox_cell = (jnp.where(sig & max0 & obj00, ciou0, 0.0)
                 + jnp.where(sig & ~max0, ciou1, 0.0))
    bbox_sum = jnp.sum(bbox_cell)

    cls_sum = jnp.sum(jnp.where(sig, bsum, 0.0))

    partial = (obj_sum + LAMBDA_NOOBJ * (cse_tot - obj_sum)
               + LAMBDA_COORD * bbox_sum + LAMBDA_CLS * cls_sum)

    prev = jnp.where(step == 0, jnp.zeros((1, 1), jnp.float32), o_ref[...])
    total = prev + partial
    o_ref[...] = jnp.where(step == GRID - 1, total / BATCH, total)


def kernel(pred, target):
    pt = jnp.transpose(pred, (1, 2, 3, 0)).reshape(CELLS, CH, BATCH)
    tt = jnp.transpose(target, (1, 2, 3, 0)).reshape(CELLS, CH, BATCH)
    out = pl.pallas_call(
        _loss_kernel,
        grid=(GRID,),
        in_specs=[
            pl.BlockSpec((CELLS, CH, BL), lambda i: (0, 0, i)),
            pl.BlockSpec((CELLS, CH, BL), lambda i: (0, 0, i)),
            pl.BlockSpec(memory_space=pl.ANY),
            pl.BlockSpec(memory_space=pl.ANY),
        ],
        out_specs=pl.BlockSpec((1, 1), lambda i: (0, 0)),
        out_shape=jax.ShapeDtypeStruct((1, 1), jnp.float32),
        scratch_shapes=([pltpu.VMEM((CELLS, BL), jnp.float32)] * 20
                        + [pltpu.SemaphoreType.DMA]),
        compiler_params=pltpu.CompilerParams(
            vmem_limit_bytes=120 * 1024 * 1024),
    )(pt, tt, pt, tt)
    return out[0, 0]
